# baseline (speedup 1.0000x reference)
import functools

import jax
import jax.numpy as jnp
from jax import lax
from jax.experimental import pallas as pl
from jax.experimental.pallas import tpu as pltpu
from jax.experimental.pallas import tpu_sc as plsc

PROB_DO = 0.5
CHUNK_ROWS = 16
TC_ROWS = 10240
TC_BLOCK = 1024


def _tc_body(c_ref, x_ref, o_ref):
    x = x_ref[...]
    o_ref[...] = jnp.maximum(x, x * c_ref[...])


def kernel(x):
    batch, width = x.shape
    info = plsc.get_sparse_core_info()
    nc, ns, lanes = info.num_cores, info.num_subcores, info.num_lanes
    nw = nc * ns
    sc_rows = batch - TC_ROWS
    rows_per_w = sc_rows // nw
    n_chunks = rows_per_w // CHUNK_ROWS
    vecs_per_row = width // lanes

    with jax.ensure_compile_time_eval():
        mask = jax.random.uniform(jax.random.key(1), (batch,)) < PROB_DO
        cf = (1.0 - mask.astype(x.dtype))[:, None]
        c_tc = cf[:TC_ROWS]
        c_sc = jnp.broadcast_to(cf[TC_ROWS:], (sc_rows, lanes))

    mesh = plsc.VectorSubcoreMesh(core_axis_name="c", subcore_axis_name="s")

    @functools.partial(
        pl.kernel,
        out_type=jax.ShapeDtypeStruct((sc_rows, width), x.dtype),
        mesh=mesh,
        scratch_types=[
            pltpu.VMEM((rows_per_w, lanes), x.dtype),
            pltpu.VMEM((CHUNK_ROWS, width), x.dtype),
            pltpu.VMEM((CHUNK_ROWS, width), x.dtype),
            pltpu.VMEM((CHUNK_ROWS, width), x.dtype),
            pltpu.VMEM((CHUNK_ROWS, width), x.dtype),
            pltpu.SemaphoreType.DMA,
            pltpu.SemaphoreType.DMA,
            pltpu.SemaphoreType.DMA,
            pltpu.SemaphoreType.DMA,
        ],
    )
    def _sc(c_hbm, x_hbm, o_hbm, c_v, i0, i1, o0, o1, si0, si1, so0, so1):
        wid = lax.axis_index("s") * nc + lax.axis_index("c")
        base = wid * rows_per_w
        pltpu.sync_copy(c_hbm.at[pl.ds(base, rows_per_w)], c_v)

        ibufs, obufs = (i0, i1), (o0, o1)
        sins, souts = (si0, si1), (so0, so1)

        def in_slice(g):
            return x_hbm.at[pl.ds(TC_ROWS + base + g * CHUNK_ROWS, CHUNK_ROWS)]

        def out_slice(g):
            return o_hbm.at[pl.ds(base + g * CHUNK_ROWS, CHUNK_ROWS)]

        pltpu.async_copy(in_slice(0), i0, si0)

        def outer(gg, _):
            for b in range(2):
                g = gg * 2 + b
                ib, ob = ibufs[b], obufs[b]

                pltpu.make_async_copy(in_slice(g), ib, sins[b]).wait()

                @pl.when(g + 1 < n_chunks)
                def _start_next_in():
                    pltpu.async_copy(in_slice(g + 1), ibufs[1 - b], sins[1 - b])

                @pl.when(g >= 2)
                def _drain_prev_out():
                    pltpu.make_async_copy(ob, out_slice(g - 2), souts[b]).wait()

                def row(r, _):
                    cv = c_v[g * CHUNK_ROWS + r, :]
                    for j in range(vecs_per_row):
                        v = ib[r, pl.ds(j * lanes, lanes)]
                        ob[r, pl.ds(j * lanes, lanes)] = jnp.maximum(v, v * cv)
                    return 0

                lax.fori_loop(0, CHUNK_ROWS, row, 0)
                pltpu.async_copy(ob, out_slice(g), souts[b])
            return 0

        lax.fori_loop(0, n_chunks // 2, outer, 0)
        pltpu.make_async_copy(o0, out_slice(n_chunks - 2), so0).wait()
        pltpu.make_async_copy(o1, out_slice(n_chunks - 1), so1).wait()

    sc_out = _sc(c_sc, x)

    tc_out = pl.pallas_call(
        _tc_body,
        grid=(TC_ROWS // TC_BLOCK,),
        in_specs=[
            pl.BlockSpec((TC_BLOCK, 1), lambda i: (i, 0)),
            pl.BlockSpec((TC_BLOCK, width), lambda i: (i, 0)),
        ],
        out_specs=pl.BlockSpec((TC_BLOCK, width), lambda i: (i, 0)),
        out_shape=jax.ShapeDtypeStruct((TC_ROWS, width), x.dtype),
    )(c_tc, x[:TC_ROWS])

    return jnp.concatenate([tc_out, sc_out], axis=0)


# speedup vs baseline: 1.6441x; 1.6441x over previous
import functools

import jax
import jax.numpy as jnp
from jax import lax
from jax.experimental import pallas as pl
from jax.experimental.pallas import tpu as pltpu
from jax.experimental.pallas import tpu_sc as plsc

PROB_DO = 0.5
CHUNK_ROWS = 16


def kernel(x):
    batch, width = x.shape
    info = plsc.get_sparse_core_info()
    nc, ns, lanes = info.num_cores, info.num_subcores, info.num_lanes
    nw = nc * ns
    rows_per_w = batch // nw
    n_chunks = rows_per_w // CHUNK_ROWS
    vecs_per_row = width // lanes

    with jax.ensure_compile_time_eval():
        mask = jax.random.uniform(jax.random.key(1), (batch,)) < PROB_DO
        c = jnp.broadcast_to(
            (1.0 - mask.astype(x.dtype))[:, None], (batch, lanes)
        )

    mesh = plsc.VectorSubcoreMesh(core_axis_name="c", subcore_axis_name="s")

    @functools.partial(
        pl.kernel,
        out_type=jax.ShapeDtypeStruct((batch, width), x.dtype),
        mesh=mesh,
        scratch_types=[
            pltpu.VMEM((rows_per_w, lanes), x.dtype),
            pltpu.VMEM((CHUNK_ROWS, width), x.dtype),
            pltpu.VMEM((CHUNK_ROWS, width), x.dtype),
            pltpu.VMEM((CHUNK_ROWS, width), x.dtype),
            pltpu.VMEM((CHUNK_ROWS, width), x.dtype),
            pltpu.SemaphoreType.DMA,
            pltpu.SemaphoreType.DMA,
            pltpu.SemaphoreType.DMA,
            pltpu.SemaphoreType.DMA,
        ],
    )
    def _sc(c_hbm, x_hbm, o_hbm, c_v, i0, i1, o0, o1, si0, si1, so0, so1):
        wid = lax.axis_index("s") * nc + lax.axis_index("c")
        base = wid * rows_per_w
        pltpu.sync_copy(c_hbm.at[pl.ds(base, rows_per_w)], c_v)

        ibufs, obufs = (i0, i1), (o0, o1)
        sins, souts = (si0, si1), (so0, so1)

        def in_slice(g):
            return x_hbm.at[pl.ds(base + g * CHUNK_ROWS, CHUNK_ROWS)]

        def out_slice(g):
            return o_hbm.at[pl.ds(base + g * CHUNK_ROWS, CHUNK_ROWS)]

        pltpu.async_copy(in_slice(0), i0, si0)

        def outer(gg, _):
            for b in range(2):
                g = gg * 2 + b
                ib, ob = ibufs[b], obufs[b]

                pltpu.make_async_copy(in_slice(g), ib, sins[b]).wait()

                @pl.when(g + 1 < n_chunks)
                def _start_next_in():
                    pltpu.async_copy(in_slice(g + 1), ibufs[1 - b], sins[1 - b])

                @pl.when(g >= 2)
                def _drain_prev_out():
                    pltpu.make_async_copy(ob, out_slice(g - 2), souts[b]).wait()

                def row(r, _):
                    cv = c_v[g * CHUNK_ROWS + r, :]
                    for j in range(vecs_per_row):
                        v = ib[r, pl.ds(j * lanes, lanes)]
                        ob[r, pl.ds(j * lanes, lanes)] = jnp.maximum(v, v * cv)
                    return 0

                lax.fori_loop(0, CHUNK_ROWS, row, 0)
                pltpu.async_copy(ob, out_slice(g), souts[b])
            return 0

        lax.fori_loop(0, n_chunks // 2, outer, 0)
        pltpu.make_async_copy(o0, out_slice(n_chunks - 2), so0).wait()
        pltpu.make_async_copy(o1, out_slice(n_chunks - 1), so1).wait()

    return _sc(c, x)


# speedup vs baseline: 1.8658x; 1.1349x over previous
import functools

import jax
import jax.numpy as jnp
from jax import lax
from jax.experimental import pallas as pl
from jax.experimental.pallas import tpu as pltpu
from jax.experimental.pallas import tpu_sc as plsc

PROB_DO = 0.5
CHUNK_ROWS = 8
DEPTH = 4


def kernel(x):
    batch, width = x.shape
    info = plsc.get_sparse_core_info()
    nc, ns, lanes = info.num_cores, info.num_subcores, info.num_lanes
    nw = nc * ns
    rows_per_w = batch // nw
    n_chunks = rows_per_w // CHUNK_ROWS
    vecs_per_row = width // lanes

    with jax.ensure_compile_time_eval():
        mask = jax.random.uniform(jax.random.key(1), (batch,)) < PROB_DO
        c = jnp.broadcast_to(
            (1.0 - mask.astype(x.dtype))[:, None], (batch, lanes)
        )

    mesh = plsc.VectorSubcoreMesh(core_axis_name="c", subcore_axis_name="s")

    @functools.partial(
        pl.kernel,
        out_type=jax.ShapeDtypeStruct((batch, width), x.dtype),
        mesh=mesh,
        scratch_types=[
            pltpu.VMEM((rows_per_w, lanes), x.dtype),
            pltpu.VMEM((DEPTH, CHUNK_ROWS, width), x.dtype),
            pltpu.VMEM((DEPTH, CHUNK_ROWS, width), x.dtype),
        ] + [pltpu.SemaphoreType.DMA] * (2 * DEPTH),
    )
    def _sc(c_hbm, x_hbm, o_hbm, c_v, ibuf, obuf, *sems):
        sins, souts = sems[:DEPTH], sems[DEPTH:]
        wid = lax.axis_index("s") * nc + lax.axis_index("c")
        base = wid * rows_per_w
        pltpu.sync_copy(c_hbm.at[pl.ds(base, rows_per_w)], c_v)

        def in_slice(g):
            return x_hbm.at[pl.ds(base + g * CHUNK_ROWS, CHUNK_ROWS)]

        def out_slice(g):
            return o_hbm.at[pl.ds(base + g * CHUNK_ROWS, CHUNK_ROWS)]

        for d in range(DEPTH - 1):
            pltpu.async_copy(in_slice(d), ibuf.at[d], sins[d])

        def outer(gg, _):
            for b in range(DEPTH):
                g = gg * DEPTH + b
                ib, ob = ibuf.at[b], obuf.at[b]

                pltpu.make_async_copy(in_slice(g), ib, sins[b]).wait()

                @pl.when(g + DEPTH - 1 < n_chunks)
                def _start_next_in():
                    nb = (b + DEPTH - 1) % DEPTH
                    pltpu.async_copy(in_slice(g + DEPTH - 1), ibuf.at[nb],
                                     sins[nb])

                @pl.when(g >= DEPTH)
                def _drain_prev_out():
                    pltpu.make_async_copy(ob, out_slice(g - DEPTH),
                                          souts[b]).wait()

                def row(r, _):
                    cv = c_v[g * CHUNK_ROWS + r, :]
                    for j in range(vecs_per_row):
                        v = ib[r, pl.ds(j * lanes, lanes)]
                        ob[r, pl.ds(j * lanes, lanes)] = jnp.maximum(v, v * cv)
                    return 0

                lax.fori_loop(0, CHUNK_ROWS, row, 0)
                pltpu.async_copy(ob, out_slice(g), souts[b])
            return 0

        lax.fori_loop(0, n_chunks // DEPTH, outer, 0)
        for d in range(DEPTH):
            g = n_chunks - DEPTH + d
            pltpu.make_async_copy(obuf.at[g % DEPTH], out_slice(g),
                                  souts[g % DEPTH]).wait()

    return _sc(c, x)


# speedup vs baseline: 1.8706x; 1.0026x over previous
import functools

import jax
import jax.numpy as jnp
from jax import lax
from jax.experimental import pallas as pl
from jax.experimental.pallas import tpu as pltpu
from jax.experimental.pallas import tpu_sc as plsc

PROB_DO = 0.5
CHUNK_ROWS = 4
DEPTH = 8


def kernel(x):
    batch, width = x.shape
    info = plsc.get_sparse_core_info()
    nc, ns, lanes = info.num_cores, info.num_subcores, info.num_lanes
    nw = nc * ns
    rows_per_w = batch // nw
    n_chunks = rows_per_w // CHUNK_ROWS
    vecs_per_row = width // lanes

    with jax.ensure_compile_time_eval():
        mask = jax.random.uniform(jax.random.key(1), (batch,)) < PROB_DO
        c = jnp.broadcast_to(
            (1.0 - mask.astype(x.dtype))[:, None], (batch, lanes)
        )

    mesh = plsc.VectorSubcoreMesh(core_axis_name="c", subcore_axis_name="s")

    @functools.partial(
        pl.kernel,
        out_type=jax.ShapeDtypeStruct((batch, width), x.dtype),
        mesh=mesh,
        scratch_types=[
            pltpu.VMEM((rows_per_w, lanes), x.dtype),
            pltpu.VMEM((DEPTH, CHUNK_ROWS, width), x.dtype),
            pltpu.VMEM((DEPTH, CHUNK_ROWS, width), x.dtype),
        ] + [pltpu.SemaphoreType.DMA] * (2 * DEPTH),
    )
    def _sc(c_hbm, x_hbm, o_hbm, c_v, ibuf, obuf, *sems):
        sins, souts = sems[:DEPTH], sems[DEPTH:]
        wid = lax.axis_index("s") * nc + lax.axis_index("c")
        base = wid * rows_per_w
        pltpu.sync_copy(c_hbm.at[pl.ds(base, rows_per_w)], c_v)

        def in_slice(g):
            return x_hbm.at[pl.ds(base + g * CHUNK_ROWS, CHUNK_ROWS)]

        def out_slice(g):
            return o_hbm.at[pl.ds(base + g * CHUNK_ROWS, CHUNK_ROWS)]

        for d in range(DEPTH - 1):
            pltpu.async_copy(in_slice(d), ibuf.at[d], sins[d])

        def outer(gg, _):
            for b in range(DEPTH):
                g = gg * DEPTH + b
                ib, ob = ibuf.at[b], obuf.at[b]

                pltpu.make_async_copy(in_slice(g), ib, sins[b]).wait()

                @pl.when(g + DEPTH - 1 < n_chunks)
                def _start_next_in():
                    nb = (b + DEPTH - 1) % DEPTH
                    pltpu.async_copy(in_slice(g + DEPTH - 1), ibuf.at[nb],
                                     sins[nb])

                @pl.when(g >= DEPTH)
                def _drain_prev_out():
                    pltpu.make_async_copy(ob, out_slice(g - DEPTH),
                                          souts[b]).wait()

                def row(r, _):
                    cv = c_v[g * CHUNK_ROWS + r, :]
                    for j in range(vecs_per_row):
                        v = ib[r, pl.ds(j * lanes, lanes)]
                        ob[r, pl.ds(j * lanes, lanes)] = jnp.maximum(v, v * cv)
                    return 0

                lax.fori_loop(0, CHUNK_ROWS, row, 0)
                pltpu.async_copy(ob, out_slice(g), souts[b])
            return 0

        lax.fori_loop(0, n_chunks // DEPTH, outer, 0)
        for d in range(DEPTH):
            g = n_chunks - DEPTH + d
            pltpu.make_async_copy(obuf.at[g % DEPTH], out_slice(g),
                                  souts[g % DEPTH]).wait()

    return _sc(c, x)
